# Initial kernel scaffold; baseline (speedup 1.0000x reference)
import jax
import jax.numpy as jnp
from jax.experimental import pallas as pl


def kernel(node_record, t_record, emb_table):
    raise NotImplementedError("write your pallas kernel here")


# speedup vs baseline: 7.7850x; 7.7850x over previous
import functools

import jax
import jax.numpy as jnp
from jax import lax
from jax.experimental import pallas as pl
from jax.experimental.pallas import tpu as pltpu
from jax.experimental.pallas import tpu_sc as plsc

NUM_LAYERS = 3
ENC_DIM = 64
N_CAT = NUM_LAYERS + 2
N_PAIR = N_CAT * N_CAT
TAB_ROWS = 32
BATCH = 4096
NEIGH = 50
TOTAL = BATCH * NEIGH

NC, NS, LANES = 2, 16, 16
NW = NC * NS
ROWS_PER_W = TOTAL // NW
SUB = 128
NSUB = 10
CHUNK = SUB * NSUB
NCHUNK = ROWS_PER_W // CHUNK


def _sc_body(node_hbm, t_hbm, emb_hbm, out_hbm, tab_hbm,
             emb_v, st_v, n_v, t_v, idx_v, rows_v, sem):
    cid = lax.axis_index("c")
    sid = lax.axis_index("s")
    wid = sid * NC + cid
    tab_base = wid * TAB_ROWS

    pltpu.sync_copy(emb_hbm, emb_v)
    for i in range(N_CAT):
        for j in range(N_CAT):
            for c in range(ENC_DIM // LANES):
                sl = pl.ds(c * LANES, LANES)
                st_v[i * N_CAT + j, sl] = emb_v[i, sl] + emb_v[j, sl]
    pltpu.sync_copy(st_v, tab_hbm.at[pl.ds(tab_base, TAB_ROWS)])

    for k in range(NCHUNK):
        blk = wid * NCHUNK + k
        pltpu.sync_copy(node_hbm.at[blk], n_v)
        pltpu.sync_copy(t_hbm.at[blk], t_v)

        for j in range(NSUB):
            def sub_body(i, _, j=j):
                off = pl.multiple_of(i * LANES, LANES)
                sl = pl.ds(off, LANES)
                n = n_v[j, sl]
                t = t_v[j, sl]
                s = n + (t * 1000.0).astype(jnp.int32)
                d1 = n - N_CAT * ((n * 26215) >> 17)
                d2 = s - N_CAT * ((s * 26215) >> 17)
                pair = d1 * N_CAT + d2
                pair = jnp.where(n == 0, N_PAIR - 1, pair)
                idx_v[j, sl] = pair + tab_base
                return 0
            lax.fori_loop(0, SUB // LANES, sub_body, 0)

        copies = [
            pltpu.async_copy(tab_hbm.at[idx_v.at[j]], rows_v.at[j], sem)
            for j in range(NSUB)
        ]
        for cp in copies:
            cp.wait()
        pltpu.sync_copy(rows_v, out_hbm.at[blk])


@functools.partial(
    pl.kernel,
    out_type=(
        jax.ShapeDtypeStruct((NW * NCHUNK, NSUB, SUB, ENC_DIM), jnp.float32),
        jax.ShapeDtypeStruct((NW * TAB_ROWS, ENC_DIM), jnp.float32),
    ),
    mesh=plsc.VectorSubcoreMesh(core_axis_name="c", subcore_axis_name="s"),
    compiler_params=pltpu.CompilerParams(use_tc_tiling_on_sc=False),
    scratch_types=[
        pltpu.VMEM((N_CAT, ENC_DIM), jnp.float32),
        pltpu.VMEM((TAB_ROWS, ENC_DIM), jnp.float32),
        pltpu.VMEM((NSUB, SUB), jnp.int32),
        pltpu.VMEM((NSUB, SUB), jnp.float32),
        pltpu.VMEM((NSUB, SUB), jnp.int32),
        pltpu.VMEM((NSUB, SUB, ENC_DIM), jnp.float32),
        pltpu.SemaphoreType.DMA,
    ],
)
def _sc_encode(node_hbm, t_hbm, emb_hbm, out_hbm, tab_hbm,
               emb_v, st_v, n_v, t_v, idx_v, rows_v, sem):
    _sc_body(node_hbm, t_hbm, emb_hbm, out_hbm, tab_hbm,
             emb_v, st_v, n_v, t_v, idx_v, rows_v, sem)


def kernel(node_record, t_record, emb_table):
    node3 = node_record.reshape(NW * NCHUNK, NSUB, SUB)
    t3 = t_record.reshape(NW * NCHUNK, NSUB, SUB)
    out, _ = _sc_encode(node3, t3, emb_table)
    return out.reshape(BATCH, NEIGH, ENC_DIM)


# speedup vs baseline: 8.2319x; 1.0574x over previous
import functools

import jax
import jax.numpy as jnp
from jax import lax
from jax.experimental import pallas as pl
from jax.experimental.pallas import tpu as pltpu
from jax.experimental.pallas import tpu_sc as plsc

NUM_LAYERS = 3
ENC_DIM = 64
N_CAT = NUM_LAYERS + 2
N_PAIR = N_CAT * N_CAT
TAB_ROWS = 32
BATCH = 4096
NEIGH = 50
TOTAL = BATCH * NEIGH

NC, NS, LANES = 2, 16, 16
NW = NC * NS
ROWS_PER_W = TOTAL // NW
SUB = 128
NSUB = 5
CHUNK = SUB * NSUB
NCHUNK = ROWS_PER_W // CHUNK


def _sc_body(node_hbm, t_hbm, emb_hbm, out_hbm, tab_hbm,
             emb_v, st_v, n_v, t_v, idx_v, rows_v,
             sem_in0, sem_in1, sem_g, sem_out):
    cid = lax.axis_index("c")
    sid = lax.axis_index("s")
    wid = sid * NC + cid
    tab_base = wid * TAB_ROWS
    sem_in = (sem_in0, sem_in1)

    pltpu.sync_copy(emb_hbm, emb_v)
    for i in range(N_CAT):
        for j in range(N_CAT):
            for c in range(ENC_DIM // LANES):
                sl = pl.ds(c * LANES, LANES)
                st_v[i * N_CAT + j, sl] = emb_v[i, sl] + emb_v[j, sl]
    pltpu.sync_copy(st_v, tab_hbm.at[pl.ds(tab_base, TAB_ROWS)])

    in_cp, out_cp = {}, {}

    def fire_in(k):
        if k < NCHUNK:
            p = k % 2
            blk = wid * NCHUNK + k
            in_cp[k] = (
                pltpu.async_copy(node_hbm.at[blk], n_v.at[p], sem_in[p]),
                pltpu.async_copy(t_hbm.at[blk], t_v.at[p], sem_in[p]),
            )

    def compute_idx(k):
        p = k % 2
        for c in in_cp.pop(k):
            c.wait()
        for j in range(NSUB):
            def sub_body(i, _, j=j, p=p):
                off = pl.multiple_of(i * LANES, LANES)
                sl = pl.ds(off, LANES)
                n = n_v[p, j, sl]
                t = t_v[p, j, sl]
                s = n + (t * 1000.0).astype(jnp.int32)
                d1 = n - N_CAT * ((n * 26215) >> 17)
                d2 = s - N_CAT * ((s * 26215) >> 17)
                pair = d1 * N_CAT + d2
                pair = jnp.where(n == 0, N_PAIR - 1, pair)
                idx_v[p, j, sl] = pair + tab_base
                return 0
            lax.fori_loop(0, SUB // LANES, sub_body, 0)

    def fire_gathers(k):
        p = k % 2
        return [
            pltpu.async_copy(tab_hbm.at[idx_v.at[p, j]], rows_v.at[p, j], sem_g)
            for j in range(NSUB)
        ]

    def fire_out(k):
        p = k % 2
        blk = wid * NCHUNK + k
        out_cp[k] = pltpu.async_copy(rows_v.at[p], out_hbm.at[blk], sem_out)

    fire_in(0)
    fire_in(1)
    compute_idx(0)
    fire_in(2)
    g = fire_gathers(0)
    for k in range(NCHUNK):
        if k + 1 < NCHUNK:
            compute_idx(k + 1)
            fire_in(k + 3)
        if k - 1 in out_cp:
            out_cp.pop(k - 1).wait()
        for cp in g:
            cp.wait()
        fire_out(k)
        if k + 1 < NCHUNK:
            g = fire_gathers(k + 1)
    out_cp.pop(NCHUNK - 1).wait()


@functools.partial(
    pl.kernel,
    out_type=(
        jax.ShapeDtypeStruct((NW * NCHUNK, NSUB, SUB, ENC_DIM), jnp.float32),
        jax.ShapeDtypeStruct((NW * TAB_ROWS, ENC_DIM), jnp.float32),
    ),
    mesh=plsc.VectorSubcoreMesh(core_axis_name="c", subcore_axis_name="s",
                                num_cores=NC, num_subcores=NS),
    compiler_params=pltpu.CompilerParams(use_tc_tiling_on_sc=False),
    scratch_types=[
        pltpu.VMEM((N_CAT, ENC_DIM), jnp.float32),
        pltpu.VMEM((TAB_ROWS, ENC_DIM), jnp.float32),
        pltpu.VMEM((2, NSUB, SUB), jnp.int32),
        pltpu.VMEM((2, NSUB, SUB), jnp.float32),
        pltpu.VMEM((2, NSUB, SUB), jnp.int32),
        pltpu.VMEM((2, NSUB, SUB, ENC_DIM), jnp.float32),
        pltpu.SemaphoreType.DMA,
        pltpu.SemaphoreType.DMA,
        pltpu.SemaphoreType.DMA,
        pltpu.SemaphoreType.DMA,
    ],
)
def _sc_encode(node_hbm, t_hbm, emb_hbm, out_hbm, tab_hbm,
               emb_v, st_v, n_v, t_v, idx_v, rows_v,
               sem_in0, sem_in1, sem_g, sem_out):
    _sc_body(node_hbm, t_hbm, emb_hbm, out_hbm, tab_hbm,
             emb_v, st_v, n_v, t_v, idx_v, rows_v,
             sem_in0, sem_in1, sem_g, sem_out)


def kernel(node_record, t_record, emb_table):
    node3 = node_record.reshape(NW * NCHUNK, NSUB, SUB)
    t3 = t_record.reshape(NW * NCHUNK, NSUB, SUB)
    out, _ = _sc_encode(node3, t3, emb_table)
    return out.reshape(BATCH, NEIGH, ENC_DIM)


# speedup vs baseline: 11.5610x; 1.4044x over previous
import functools

import jax
import jax.numpy as jnp
from jax import lax
from jax.experimental import pallas as pl
from jax.experimental.pallas import tpu as pltpu
from jax.experimental.pallas import tpu_sc as plsc

NUM_LAYERS = 3
ENC_DIM = 64
N_CAT = NUM_LAYERS + 2
N_PAIR = N_CAT * N_CAT
BATCH = 4096
NEIGH = 50
NPAD = 56

NC, NS, LANES = 2, 16, 16
NW = NC * NS
BBLK = BATCH // NW
CODE_PITCH = 128


def _sc_body(node_hbm, t_hbm, emb_hbm, out_hbm,
             emb_v, stT, n_v, t_v, stage, sem_out0, sem_out1):
    cid = lax.axis_index("c")
    sid = lax.axis_index("s")
    wid = sid * NC + cid
    wb = pl.multiple_of(wid * BBLK, BBLK)
    sems = (sem_out0, sem_out1)

    pltpu.sync_copy(emb_hbm, emb_v)
    iota = lax.iota(jnp.int32, LANES)
    for dc in range(ENC_DIM // LANES):
        dsl = pl.ds(dc * LANES, LANES)
        base16 = (iota + dc * LANES) * CODE_PITCH
        for i in range(N_CAT):
            for j in range(N_CAT):
                val = emb_v[i, dsl] + emb_v[j, dsl]
                plsc.store_scatter(stT, [base16 + (i * N_CAT + j)], val)

    pltpu.sync_copy(node_hbm.at[:, pl.ds(wb, BBLK)], n_v)
    pltpu.sync_copy(t_hbm.at[:, pl.ds(wb, BBLK)], t_v)

    def emit_n(n, p):
        for bc in range(BBLK // LANES):
            sl = pl.ds(bc * LANES, LANES)
            nn = n_v[n, sl]
            tt = t_v[n, sl]
            s = nn + (tt * 1000.0).astype(jnp.int32)
            d1 = nn - N_CAT * ((nn * 26215) >> 17)
            d2 = s - N_CAT * ((s * 26215) >> 17)
            code = jnp.where(nn == 0, N_PAIR - 1, d1 * N_CAT + d2)
            for d in range(ENC_DIM):
                stage[p, d, sl] = plsc.load_gather(stT, [code + d * CODE_PITCH])
        pltpu.async_copy(stage.at[p], out_hbm.at[n, :, pl.ds(wb, BBLK)],
                         sems[p])

    def loop_body(i, carry):
        for p in range(2):
            @pl.when(i >= 1)
            def _(p=p):
                pltpu.make_async_copy(
                    stage.at[p], out_hbm.at[0, :, pl.ds(wb, BBLK)], sems[p]
                ).wait()
            emit_n(2 * i + p, p)
        return carry

    lax.fori_loop(0, NEIGH // 2, loop_body, 0)
    for p in range(2):
        pltpu.make_async_copy(
            stage.at[p], out_hbm.at[0, :, pl.ds(wb, BBLK)], sems[p]
        ).wait()


@functools.partial(
    pl.kernel,
    out_type=jax.ShapeDtypeStruct((NEIGH, ENC_DIM, BATCH), jnp.float32),
    mesh=plsc.VectorSubcoreMesh(core_axis_name="c", subcore_axis_name="s",
                                num_cores=NC, num_subcores=NS),
    compiler_params=pltpu.CompilerParams(use_tc_tiling_on_sc=True,
                                         needs_layout_passes=False),
    scratch_types=[
        pltpu.VMEM((N_CAT, ENC_DIM), jnp.float32),
        pltpu.VMEM((ENC_DIM * CODE_PITCH,), jnp.float32),
        pltpu.VMEM((NPAD, BBLK), jnp.int32),
        pltpu.VMEM((NPAD, BBLK), jnp.float32),
        pltpu.VMEM((2, ENC_DIM, BBLK), jnp.float32),
        pltpu.SemaphoreType.DMA,
        pltpu.SemaphoreType.DMA,
    ],
)
def _sc_encode(node_hbm, t_hbm, emb_hbm, out_hbm,
               emb_v, stT, n_v, t_v, stage, sem_out0, sem_out1):
    _sc_body(node_hbm, t_hbm, emb_hbm, out_hbm,
             emb_v, stT, n_v, t_v, stage, sem_out0, sem_out1)


def kernel(node_record, t_record, emb_table):
    nodeT = jnp.pad(node_record.transpose(1, 0), ((0, NPAD - NEIGH), (0, 0)))
    tT = jnp.pad(t_record.transpose(1, 0), ((0, NPAD - NEIGH), (0, 0)))
    outT = _sc_encode(nodeT, tT, emb_table)
    return outT.transpose(2, 0, 1)


# speedup vs baseline: 38.8463x; 3.3601x over previous
import functools

import jax
import jax.numpy as jnp
from jax import lax
from jax.experimental import pallas as pl
from jax.experimental.pallas import tpu as pltpu
from jax.experimental.pallas import tpu_sc as plsc

NUM_LAYERS = 3
ENC_DIM = 64
N_CAT = NUM_LAYERS + 2
N_PAIR = N_CAT * N_CAT
BATCH = 4096
NEIGH = 50
NPAD = 56

NC, NS, LANES = 2, 16, 16
NW = NC * NS
BBLK = BATCH // NW
CODE_PITCH = 128


def _sc_body(node_hbm, t_hbm, emb_hbm, out_hbm,
             emb_v, stT, n_v, t_v, stage, sem_out0, sem_out1):
    cid = lax.axis_index("c")
    sid = lax.axis_index("s")
    wid = sid * NC + cid
    wb = pl.multiple_of(wid * BBLK, BBLK)
    sems = (sem_out0, sem_out1)

    pltpu.sync_copy(emb_hbm, emb_v)
    iota = lax.iota(jnp.int32, LANES)
    for dc in range(ENC_DIM // LANES):
        dsl = pl.ds(dc * LANES, LANES)
        base16 = (iota + dc * LANES) * CODE_PITCH
        for i in range(N_CAT):
            for j in range(N_CAT):
                val = emb_v[i, dsl] + emb_v[j, dsl]
                plsc.store_scatter(stT, [base16 + (i * N_CAT + j)], val)

    pltpu.sync_copy(node_hbm.at[:, pl.ds(wb, BBLK)], n_v)
    pltpu.sync_copy(t_hbm.at[:, pl.ds(wb, BBLK)], t_v)

    def emit_n(n, p):
        for bc in range(BBLK // LANES):
            sl = pl.ds(bc * LANES, LANES)
            nn = n_v[n, sl]
            tt = t_v[n, sl]
            s = nn + (tt * 1000.0).astype(jnp.int32)
            d1 = nn - N_CAT * ((nn * 26215) >> 17)
            d2 = s - N_CAT * ((s * 26215) >> 17)
            code = jnp.where(nn == 0, N_PAIR - 1, d1 * N_CAT + d2)

            @plsc.parallel_loop(0, ENC_DIM, unroll=8)
            def _(d, code=code, sl=sl, p=p):
                stage[p, d, sl] = plsc.load_gather(stT, [code + d * CODE_PITCH])
        pltpu.async_copy(stage.at[p], out_hbm.at[n, :, pl.ds(wb, BBLK)],
                         sems[p])

    def loop_body(i, carry):
        for p in range(2):
            @pl.when(i >= 1)
            def _(p=p):
                pltpu.make_async_copy(
                    stage.at[p], out_hbm.at[0, :, pl.ds(wb, BBLK)], sems[p]
                ).wait()
            emit_n(2 * i + p, p)
        return carry

    lax.fori_loop(0, NEIGH // 2, loop_body, 0)
    for p in range(2):
        pltpu.make_async_copy(
            stage.at[p], out_hbm.at[0, :, pl.ds(wb, BBLK)], sems[p]
        ).wait()


@functools.partial(
    pl.kernel,
    out_type=jax.ShapeDtypeStruct((NEIGH, ENC_DIM, BATCH), jnp.float32),
    mesh=plsc.VectorSubcoreMesh(core_axis_name="c", subcore_axis_name="s",
                                num_cores=NC, num_subcores=NS),
    compiler_params=pltpu.CompilerParams(use_tc_tiling_on_sc=True,
                                         needs_layout_passes=False),
    scratch_types=[
        pltpu.VMEM((N_CAT, ENC_DIM), jnp.float32),
        pltpu.VMEM((ENC_DIM * CODE_PITCH,), jnp.float32),
        pltpu.VMEM((NPAD, BBLK), jnp.int32),
        pltpu.VMEM((NPAD, BBLK), jnp.float32),
        pltpu.VMEM((2, ENC_DIM, BBLK), jnp.float32),
        pltpu.SemaphoreType.DMA,
        pltpu.SemaphoreType.DMA,
    ],
)
def _sc_encode(node_hbm, t_hbm, emb_hbm, out_hbm,
               emb_v, stT, n_v, t_v, stage, sem_out0, sem_out1):
    _sc_body(node_hbm, t_hbm, emb_hbm, out_hbm,
             emb_v, stT, n_v, t_v, stage, sem_out0, sem_out1)


def kernel(node_record, t_record, emb_table):
    nodeT = jnp.pad(node_record.transpose(1, 0), ((0, NPAD - NEIGH), (0, 0)))
    tT = jnp.pad(t_record.transpose(1, 0), ((0, NPAD - NEIGH), (0, 0)))
    outT = _sc_encode(nodeT, tT, emb_table)
    return outT.transpose(2, 0, 1)
